# baseline (speedup 1.0000x reference)
import functools

import jax
import jax.numpy as jnp
from jax import lax
from jax.experimental import pallas as pl
from jax.experimental.pallas import tpu as pltpu
from jax.experimental.pallas import tpu_sc as plsc

N = 10000
E = 320000
D = 128
NP = 10240
CH = 125
CHP = 128
NCHT = E // CH
NW = 32
NCHW = NCHT // NW
EPT = E // 16
RPT = NP // 16
HALF = NP // 2
RPP = HALF // 16

_mesh = plsc.VectorSubcoreMesh(core_axis_name="c", subcore_axis_name="s")
_f32 = jnp.float32


def _zero_1d(ref, n):
    def body(i, _):
        ref[pl.ds(i * 16, 16)] = jnp.zeros((16,), _f32)
        return 0
    lax.fori_loop(0, n // 16, body, 0)


def _rsqrt16(d):
    bi = lax.bitcast_convert_type(d, jnp.int32)
    y = lax.bitcast_convert_type(
        jnp.int32(0x5F3759DF) - lax.shift_right_arithmetic(bi, 1), _f32)
    for _ in range(4):
        y = y * (1.5 - 0.5 * d * y * y)
    return jnp.where(d > 0.5, y, 0.0)


@functools.partial(
    pl.kernel,
    mesh=_mesh,
    compiler_params=pltpu.CompilerParams(needs_layout_passes=False),
    out_type=jax.ShapeDtypeStruct((NP,), _f32),
    scratch_types=[
        pltpu.VMEM((EPT,), jnp.int32),
        pltpu.VMEM((NP,), _f32),
        pltpu.VMEM((RPT,), _f32),
        pltpu.VMEM((RPT,), _f32),
        pltpu.VMEM((RPT,), _f32),
        pltpu.VMEM_SHARED((16, NP), _f32),
    ],
)
def _deg(dst_hbm, dinv_hbm, dst_v, deg_v, acc_v, tmp_v, dinv_v, shared):
    c = lax.axis_index("c")
    s = lax.axis_index("s")

    _zero_1d(deg_v, NP)
    pltpu.sync_copy(dst_hbm.at[pl.ds(s * EPT, EPT)], dst_v)
    ones = jnp.ones((16,), _f32)

    def deg_body(i, _):
        idx = dst_v[pl.ds(i * 16, 16)]
        plsc.addupdate_scatter(deg_v, [idx], ones)
        return 0
    lax.fori_loop(0, EPT // 16, deg_body, 0)

    pltpu.sync_copy(deg_v, shared.at[s])
    plsc.subcore_barrier()

    base = s * RPT
    _zero_1d(acc_v, RPT)
    for k in range(16):
        pltpu.sync_copy(shared.at[k, pl.ds(base, RPT)], tmp_v)

        def add_body(i, _):
            acc_v[pl.ds(i * 16, 16)] = (acc_v[pl.ds(i * 16, 16)]
                                        + tmp_v[pl.ds(i * 16, 16)])
            return 0
        lax.fori_loop(0, RPT // 16, add_body, 0)

    def rsqrt_body(i, _):
        dinv_v[pl.ds(i * 16, 16)] = _rsqrt16(acc_v[pl.ds(i * 16, 16)])
        return 0
    lax.fori_loop(0, RPT // 16, rsqrt_body, 0)

    @pl.when(c == 0)
    def _():
        pltpu.sync_copy(dinv_v, dinv_hbm.at[pl.ds(base, RPT)])


@functools.partial(
    pl.kernel,
    mesh=_mesh,
    out_type=jax.ShapeDtypeStruct((2, NP, D), _f32),
    scratch_types=[
        pltpu.VMEM((NCHW, CHP), jnp.int32),
        pltpu.VMEM((2, NCHW, CHP), jnp.int32),
        pltpu.VMEM((2, CHP, D), _f32),
        pltpu.VMEM_SHARED((HALF + 8, D), _f32),
        pltpu.VMEM((NW, 16), jnp.int32),
        pltpu.SemaphoreType.DMA,
        pltpu.SemaphoreType.DMA,
    ],
)
def _layer(z_hbm, src_hbm, dst_hbm, bnd_hbm, p_hbm, src_v, dst_v, rows_v,
           accum, bnd_v, gsem0, gsem1):
    c = lax.axis_index("c")
    s = lax.axis_index("s")
    w = c * 16 + s
    sems = (gsem0, gsem1)

    pltpu.sync_copy(src_hbm.at[pl.ds(w * NCHW, NCHW), :], src_v)
    pltpu.sync_copy(dst_hbm.at[:, pl.ds(w * NCHW, NCHW), :], dst_v)
    pltpu.sync_copy(bnd_hbm, bnd_v)

    brow = bnd_v[w, :]
    hi0 = brow[0]
    lo1 = brow[1]

    for p in range(2):
        lo = jnp.int32(0) if p == 0 else lo1
        hi = hi0 if p == 0 else jnp.int32(NCHW)

        def zrow(i, _):
            for v in range(D // 16):
                rows_v[0, i, pl.ds(v * 16, 16)] = jnp.zeros((16,), _f32)
            return 0
        lax.fori_loop(0, CHP, zrow, 0)
        pltpu.sync_copy(rows_v.at[0], accum.at[pl.ds(s * RPP, 128), :])
        pltpu.sync_copy(rows_v.at[0], accum.at[pl.ds(s * RPP + 128, 128), :])
        pltpu.sync_copy(rows_v.at[0, pl.ds(0, 64), :],
                        accum.at[pl.ds(s * RPP + 256, 64), :])
        plsc.subcore_barrier()

        @pl.when(lo < hi)
        def _():
            pltpu.async_copy(z_hbm.at[src_v.at[lo]], rows_v.at[0], sems[0])

        @pl.when(lo + 1 < hi)
        def _():
            pltpu.async_copy(z_hbm.at[src_v.at[lo + 1]], rows_v.at[1],
                             sems[1])

        def chunk_body(i, _):
            ci = lo + i
            for b in range(2):
                @pl.when(i % 2 == b)
                def _():
                    buf = rows_v.at[b]
                    pltpu.make_async_copy(
                        z_hbm.at[src_v.at[ci]], buf, sems[b]).wait()
                    pltpu.sync_copy(buf, accum.at[dst_v.at[p, ci]],
                                    add=True)

                    @pl.when(ci + 2 < hi)
                    def _():
                        pltpu.async_copy(
                            z_hbm.at[src_v.at[ci + 2]], buf, sems[b])
            return 0
        lax.fori_loop(0, hi - lo, chunk_body, 0)
        plsc.subcore_barrier()

        pltpu.sync_copy(accum.at[pl.ds(s * RPP, RPP), :],
                        p_hbm.at[c, pl.ds(p * HALF + s * RPP, RPP), :])


def _scale_body(d_ref, x_ref, o_ref):
    o_ref[...] = d_ref[...] * x_ref[...]


def _scale(d, x):
    blk = 256
    row = pl.BlockSpec((blk, D), lambda i: (i, 0))
    dsp = pl.BlockSpec((blk, 1), lambda i: (i, 0))
    return pl.pallas_call(
        _scale_body,
        grid=(NP // blk,),
        in_specs=[dsp, row],
        out_specs=row,
        out_shape=jax.ShapeDtypeStruct((NP, D), _f32),
    )(d, x)


def _comb_body(d_ref, a_ref, b_ref, x1_ref, z2_ref):
    dd = d_ref[...]
    x1 = dd * (a_ref[...] + b_ref[...])
    x1_ref[...] = x1
    z2_ref[...] = dd * x1


def _comb(d, pa, pb):
    blk = 256
    row = pl.BlockSpec((blk, D), lambda i: (i, 0))
    dsp = pl.BlockSpec((blk, 1), lambda i: (i, 0))
    return pl.pallas_call(
        _comb_body,
        grid=(NP // blk,),
        in_specs=[dsp, row, row],
        out_specs=[row, row],
        out_shape=[jax.ShapeDtypeStruct((NP, D), _f32),
                   jax.ShapeDtypeStruct((NP, D), _f32)],
    )(d, pa, pb)


def _final_body(x_ref, x1_ref, a_ref, b_ref, d_ref, wt_ref, bias_ref, o_ref):
    x2 = d_ref[...] * (a_ref[...] + b_ref[...])
    acc = (x_ref[...] + x1_ref[...] + x2) * 0.3
    o_ref[...] = (jnp.dot(acc, wt_ref[...], preferred_element_type=_f32)
                  + bias_ref[...])


def _final(x, x1, pa, pb, d, wt, bias):
    blk = 256
    row = pl.BlockSpec((blk, D), lambda i: (i, 0))
    dsp = pl.BlockSpec((blk, 1), lambda i: (i, 0))
    return pl.pallas_call(
        _final_body,
        grid=(NP // blk,),
        in_specs=[row, row, row, row, dsp,
                  pl.BlockSpec((D, D), lambda i: (0, 0)),
                  pl.BlockSpec((1, D), lambda i: (0, 0))],
        out_specs=row,
        out_shape=jax.ShapeDtypeStruct((NP, D), _f32),
    )(x, x1, pa, pb, d, wt, bias)


def kernel(node_emb, edge_index, W, b):
    src = edge_index[0].astype(jnp.int32)
    dst = edge_index[1].astype(jnp.int32)

    mask = dst < HALF
    csum = jnp.cumsum(mask.astype(jnp.int32))
    n0 = csum[E - 1]
    rank1 = jnp.arange(1, E + 1, dtype=jnp.int32) - csum
    pos = jnp.where(mask, csum - 1, n0 + rank1 - 1)
    packed = jnp.zeros((E,), jnp.int32).at[pos].set(src * 16384 + dst)
    srcs = packed // 16384
    dsts = packed % 16384

    srcp = jnp.pad(srcs.reshape(NCHT, CH), ((0, 0), (0, CHP - CH)))
    d0 = jnp.where(dsts < HALF, dsts, HALF)
    d1 = jnp.where(dsts >= HALF, dsts - HALF, HALF)
    dstp = jnp.pad(jnp.stack([d0, d1]).reshape(2, NCHT, CH),
                   ((0, 0), (0, 0), (0, CHP - CH)), constant_values=HALF)

    b0 = (n0 + CH - 1) // CH
    f1 = n0 // CH
    wid = jnp.arange(NW, dtype=jnp.int32)
    hi0 = jnp.clip(b0 - NCHW * wid, 0, NCHW)
    lo1 = jnp.clip(f1 - NCHW * wid, 0, NCHW)
    bounds = jnp.pad(jnp.stack([hi0, lo1], axis=1),
                     ((0, 0), (0, 14))).astype(jnp.int32)

    x_pad = jnp.pad(node_emb, ((0, NP - N), (0, 0)))

    dinv = _deg(dst)
    d2 = dinv.reshape(NP, 1)
    z = _scale(d2, x_pad)
    p1 = _layer(z, srcp, dstp, bounds)
    x1, z2 = _comb(d2, p1[0], p1[1])
    p2 = _layer(z2, srcp, dstp, bounds)
    out = _final(x_pad, x1, p2[0], p2[1], d2, W.T, b.reshape(1, D))
    return out[:N]


# speedup vs baseline: 3.1350x; 3.1350x over previous
import functools

import jax
import jax.numpy as jnp
from jax import lax
from jax.experimental import pallas as pl
from jax.experimental.pallas import tpu as pltpu
from jax.experimental.pallas import tpu_sc as plsc

N = 10000
E = 320000
D = 128
NP = 10240
CHP = 128
NW = 32
NCHW = 80
NCHT = NW * NCHW
EPW = E // NW
SLOTS = NCHW * CHP
EPT = E // 16
RPT = NP // 16
HALF = NP // 2
RPP = HALF // 16

_mesh = plsc.VectorSubcoreMesh(core_axis_name="c", subcore_axis_name="s")
_f32 = jnp.float32


def _zero_1d(ref, n):
    def body(i, _):
        ref[pl.ds(i * 16, 16)] = jnp.zeros((16,), _f32)
        return 0
    lax.fori_loop(0, n // 16, body, 0)


def _rsqrt16(d):
    bi = lax.bitcast_convert_type(d, jnp.int32)
    y = lax.bitcast_convert_type(
        jnp.int32(0x5F3759DF) - lax.shift_right_arithmetic(bi, 1), _f32)
    for _ in range(4):
        y = y * (1.5 - 0.5 * d * y * y)
    return jnp.where(d > 0.5, y, 0.0)


@functools.partial(
    pl.kernel,
    mesh=_mesh,
    compiler_params=pltpu.CompilerParams(needs_layout_passes=False),
    out_type=jax.ShapeDtypeStruct((NP,), _f32),
    scratch_types=[
        pltpu.VMEM((EPT,), jnp.int32),
        pltpu.VMEM((NP,), _f32),
        pltpu.VMEM((RPT,), _f32),
        pltpu.VMEM((RPT,), _f32),
        pltpu.VMEM((RPT,), _f32),
        pltpu.VMEM_SHARED((16, NP), _f32),
    ],
)
def _deg(dst_hbm, dinv_hbm, dst_v, deg_v, acc_v, tmp_v, dinv_v, shared):
    c = lax.axis_index("c")
    s = lax.axis_index("s")

    _zero_1d(deg_v, NP)
    pltpu.sync_copy(dst_hbm.at[pl.ds(s * EPT, EPT)], dst_v)
    ones = jnp.ones((16,), _f32)

    def deg_body(i, _):
        idx = dst_v[pl.ds(i * 16, 16)]
        plsc.addupdate_scatter(deg_v, [idx], ones)
        return 0
    lax.fori_loop(0, EPT // 16, deg_body, 0)

    pltpu.sync_copy(deg_v, shared.at[s])
    plsc.subcore_barrier()

    base = s * RPT
    _zero_1d(acc_v, RPT)
    for k in range(16):
        pltpu.sync_copy(shared.at[k, pl.ds(base, RPT)], tmp_v)

        def add_body(i, _):
            acc_v[pl.ds(i * 16, 16)] = (acc_v[pl.ds(i * 16, 16)]
                                        + tmp_v[pl.ds(i * 16, 16)])
            return 0
        lax.fori_loop(0, RPT // 16, add_body, 0)

    def rsqrt_body(i, _):
        dinv_v[pl.ds(i * 16, 16)] = _rsqrt16(acc_v[pl.ds(i * 16, 16)])
        return 0
    lax.fori_loop(0, RPT // 16, rsqrt_body, 0)

    @pl.when(c == 0)
    def _():
        pltpu.sync_copy(dinv_v, dinv_hbm.at[pl.ds(base, RPT)])


@functools.partial(
    pl.kernel,
    mesh=_mesh,
    compiler_params=pltpu.CompilerParams(needs_layout_passes=False),
    out_type=[
        jax.ShapeDtypeStruct((2, NCHT * CHP), jnp.int32),
        jax.ShapeDtypeStruct((2, NCHT * CHP), jnp.int32),
        jax.ShapeDtypeStruct((NW, 16), jnp.int32),
    ],
    scratch_types=[
        pltpu.VMEM((EPW,), jnp.int32),
        pltpu.VMEM((EPW,), jnp.int32),
        pltpu.VMEM((2 * SLOTS,), jnp.int32),
        pltpu.VMEM((2 * SLOTS,), jnp.int32),
        pltpu.VMEM((16,), jnp.int32),
    ],
)
def _prep(src_hbm, dst_hbm, srcp_hbm, dstp_hbm, bnd_hbm,
          src_v, dst_v, csrc_v, cdst_v, brow_v):
    c = lax.axis_index("c")
    s = lax.axis_index("s")
    w = c * 16 + s

    pltpu.sync_copy(src_hbm.at[pl.ds(w * EPW, EPW)], src_v)
    pltpu.sync_copy(dst_hbm.at[pl.ds(w * EPW, EPW)], dst_v)

    zeros16 = jnp.zeros((16,), jnp.int32)
    trash16 = jnp.full((16,), HALF, jnp.int32)

    def fill(i, _):
        csrc_v[pl.ds(i * 16, 16)] = zeros16
        cdst_v[pl.ds(i * 16, 16)] = trash16
        return 0
    lax.fori_loop(0, 2 * SLOTS // 16, fill, 0)

    i16 = lax.iota(jnp.int32, 16)

    def body(g, carry):
        off0, off1 = carry
        sv = src_v[pl.ds(g * 16, 16)]
        dv = dst_v[pl.ds(g * 16, 16)]
        m0 = dv < HALF
        inc = jnp.where(m0, 1, 0).astype(jnp.int32)
        csum = plsc.cumsum(inc)
        excl0 = csum - inc
        cnt = plsc.all_reduce_population_count(m0)[0]
        pos = jnp.where(m0, off0 + excl0, SLOTS + off1 + (i16 - excl0))
        plsc.store_scatter(csrc_v, [pos], sv)
        plsc.store_scatter(cdst_v, [pos], jnp.where(m0, dv, dv - HALF))
        return (off0 + cnt, off1 + (16 - cnt))

    off0, off1 = lax.fori_loop(0, EPW // 16, body,
                               (jnp.int32(0), jnp.int32(0)))

    for p in range(2):
        pltpu.sync_copy(csrc_v.at[pl.ds(p * SLOTS, SLOTS)],
                        srcp_hbm.at[p, pl.ds(w * SLOTS, SLOTS)])
        pltpu.sync_copy(cdst_v.at[pl.ds(p * SLOTS, SLOTS)],
                        dstp_hbm.at[p, pl.ds(w * SLOTS, SLOTS)])

    lanes = lax.iota(jnp.int32, 16)
    h0 = (jnp.full((16,), off0, jnp.int32) + CHP - 1) // CHP
    h1 = (jnp.full((16,), off1, jnp.int32) + CHP - 1) // CHP
    brow_v[pl.ds(0, 16)] = jnp.where(lanes == 0, h0,
                                     jnp.where(lanes == 1, h1, 0))
    pltpu.sync_copy(brow_v, bnd_hbm.at[w])


@functools.partial(
    pl.kernel,
    mesh=_mesh,
    out_type=jax.ShapeDtypeStruct((2, NP, D), _f32),
    scratch_types=[
        pltpu.VMEM((2, NCHW, CHP), jnp.int32),
        pltpu.VMEM((2, NCHW, CHP), jnp.int32),
        pltpu.VMEM((2, CHP, D), _f32),
        pltpu.VMEM_SHARED((HALF + 8, D), _f32),
        pltpu.VMEM((NW, 16), jnp.int32),
        pltpu.SemaphoreType.DMA,
        pltpu.SemaphoreType.DMA,
    ],
)
def _layer(z_hbm, src_hbm, dst_hbm, bnd_hbm, p_hbm, src_v, dst_v, rows_v,
           accum, bnd_v, gsem0, gsem1):
    c = lax.axis_index("c")
    s = lax.axis_index("s")
    w = c * 16 + s
    sems = (gsem0, gsem1)

    pltpu.sync_copy(src_hbm.at[:, pl.ds(w * NCHW, NCHW), :], src_v)
    pltpu.sync_copy(dst_hbm.at[:, pl.ds(w * NCHW, NCHW), :], dst_v)
    pltpu.sync_copy(bnd_hbm, bnd_v)

    brow = bnd_v[w, :]
    hi0 = brow[0]
    hi1 = brow[1]

    for p in range(2):
        lo = jnp.int32(0)
        hi = hi0 if p == 0 else hi1

        def zrow(i, _):
            for v in range(D // 16):
                rows_v[0, i, pl.ds(v * 16, 16)] = jnp.zeros((16,), _f32)
            return 0
        lax.fori_loop(0, CHP, zrow, 0)
        pltpu.sync_copy(rows_v.at[0], accum.at[pl.ds(s * RPP, 128), :])
        pltpu.sync_copy(rows_v.at[0], accum.at[pl.ds(s * RPP + 128, 128), :])
        pltpu.sync_copy(rows_v.at[0, pl.ds(0, 64), :],
                        accum.at[pl.ds(s * RPP + 256, 64), :])
        plsc.subcore_barrier()

        @pl.when(lo < hi)
        def _():
            pltpu.async_copy(z_hbm.at[src_v.at[p, lo]], rows_v.at[0],
                             sems[0])

        @pl.when(lo + 1 < hi)
        def _():
            pltpu.async_copy(z_hbm.at[src_v.at[p, lo + 1]], rows_v.at[1],
                             sems[1])

        def chunk_body(i, _):
            ci = lo + i
            for b in range(2):
                @pl.when(i % 2 == b)
                def _():
                    buf = rows_v.at[b]
                    pltpu.make_async_copy(
                        z_hbm.at[src_v.at[p, ci]], buf, sems[b]).wait()
                    pltpu.sync_copy(buf, accum.at[dst_v.at[p, ci]],
                                    add=True)

                    @pl.when(ci + 2 < hi)
                    def _():
                        pltpu.async_copy(
                            z_hbm.at[src_v.at[p, ci + 2]], buf, sems[b])
            return 0
        lax.fori_loop(0, hi - lo, chunk_body, 0)
        plsc.subcore_barrier()

        pltpu.sync_copy(accum.at[pl.ds(s * RPP, RPP), :],
                        p_hbm.at[c, pl.ds(p * HALF + s * RPP, RPP), :])


def _scale_body(d_ref, x_ref, o_ref):
    o_ref[...] = d_ref[...] * x_ref[...]


def _scale(d, x):
    blk = 256
    row = pl.BlockSpec((blk, D), lambda i: (i, 0))
    dsp = pl.BlockSpec((blk, 1), lambda i: (i, 0))
    return pl.pallas_call(
        _scale_body,
        grid=(NP // blk,),
        in_specs=[dsp, row],
        out_specs=row,
        out_shape=jax.ShapeDtypeStruct((NP, D), _f32),
    )(d, x)


def _comb_body(d_ref, a_ref, b_ref, x1_ref, z2_ref):
    dd = d_ref[...]
    x1 = dd * (a_ref[...] + b_ref[...])
    x1_ref[...] = x1
    z2_ref[...] = dd * x1


def _comb(d, pa, pb):
    blk = 256
    row = pl.BlockSpec((blk, D), lambda i: (i, 0))
    dsp = pl.BlockSpec((blk, 1), lambda i: (i, 0))
    return pl.pallas_call(
        _comb_body,
        grid=(NP // blk,),
        in_specs=[dsp, row, row],
        out_specs=[row, row],
        out_shape=[jax.ShapeDtypeStruct((NP, D), _f32),
                   jax.ShapeDtypeStruct((NP, D), _f32)],
    )(d, pa, pb)


def _final_body(x_ref, x1_ref, a_ref, b_ref, d_ref, wt_ref, bias_ref, o_ref):
    x2 = d_ref[...] * (a_ref[...] + b_ref[...])
    acc = (x_ref[...] + x1_ref[...] + x2) * 0.3
    o_ref[...] = (jnp.dot(acc, wt_ref[...], preferred_element_type=_f32)
                  + bias_ref[...])


def _final(x, x1, pa, pb, d, wt, bias):
    blk = 256
    row = pl.BlockSpec((blk, D), lambda i: (i, 0))
    dsp = pl.BlockSpec((blk, 1), lambda i: (i, 0))
    return pl.pallas_call(
        _final_body,
        grid=(NP // blk,),
        in_specs=[row, row, row, row, dsp,
                  pl.BlockSpec((D, D), lambda i: (0, 0)),
                  pl.BlockSpec((1, D), lambda i: (0, 0))],
        out_specs=row,
        out_shape=jax.ShapeDtypeStruct((NP, D), _f32),
    )(x, x1, pa, pb, d, wt, bias)


def kernel(node_emb, edge_index, W, b):
    src = edge_index[0].astype(jnp.int32)
    dst = edge_index[1].astype(jnp.int32)

    srcp_f, dstp_f, bounds = _prep(src, dst)
    srcp = srcp_f.reshape(2, NCHT, CHP)
    dstp = dstp_f.reshape(2, NCHT, CHP)

    x_pad = jnp.pad(node_emb, ((0, NP - N), (0, 0)))

    dinv = _deg(dst)
    d2 = dinv.reshape(NP, 1)
    z = _scale(d2, x_pad)
    p1 = _layer(z, srcp, dstp, bounds)
    x1, z2 = _comb(d2, p1[0], p1[1])
    p2 = _layer(z2, srcp, dstp, bounds)
    out = _final(x_pad, x1, p2[0], p2[1], d2, W.T, b.reshape(1, D))
    return out[:N]
